# Initial kernel scaffold; baseline (speedup 1.0000x reference)
import jax
import jax.numpy as jnp
from jax.experimental import pallas as pl


def kernel(s, prog, emb, lib_gain, lib_bias):
    raise NotImplementedError("write your pallas kernel here")


# speedup vs baseline: 18.1502x; 18.1502x over previous
import functools

import jax
import jax.numpy as jnp
from jax.experimental import pallas as pl
from jax.experimental.pallas import tpu as pltpu

_BLOCK_B = 2048


def _body(s_ref, prog_ref, emb_ref, gain_ref, bias_ref, o_ref, coef_ref):
    @pl.when(pl.program_id(0) == 0)
    def _compute_coefficients():
        w = jax.nn.softmax(prog_ref[...], axis=-1)
        gain = jnp.dot(w, gain_ref[...],
                       preferred_element_type=jnp.float32)
        bias = jnp.dot(w, bias_ref[...],
                       preferred_element_type=jnp.float32)
        c = gain * emb_ref[...] + bias
        P_ = gain.shape[0]
        G = gain[0:1, :]
        C = c[0:1, :]
        for i in range(1, P_):
            g = gain[i:i + 1, :]
            G = g * G
            C = g * C + c[i:i + 1, :]
        coef_ref[0:1, :] = G
        coef_ref[1:2, :] = C

    o_ref[...] = s_ref[...] * coef_ref[0:1, :] + coef_ref[1:2, :]


@jax.jit
def kernel(s, prog, emb, lib_gain, lib_bias):
    B, SD = s.shape
    P_, K = prog.shape
    emb_p = emb[:P_]
    grid = (B // _BLOCK_B,)
    return pl.pallas_call(
        _body,
        grid=grid,
        in_specs=[
            pl.BlockSpec((_BLOCK_B, SD), lambda i: (i, 0)),
            pl.BlockSpec((P_, K), lambda i: (0, 0)),
            pl.BlockSpec((P_, SD), lambda i: (0, 0)),
            pl.BlockSpec((K, SD), lambda i: (0, 0)),
            pl.BlockSpec((K, SD), lambda i: (0, 0)),
        ],
        out_specs=pl.BlockSpec((_BLOCK_B, SD), lambda i: (i, 0)),
        out_shape=jax.ShapeDtypeStruct((B, SD), s.dtype),
        scratch_shapes=[pltpu.VMEM((2, SD), jnp.float32)],
    )(s, prog, emb_p, lib_gain, lib_bias)


# speedup vs baseline: 21.5664x; 1.1882x over previous
import functools

import jax
import jax.numpy as jnp
from jax.experimental import pallas as pl
from jax.experimental.pallas import tpu as pltpu

_BLOCK_B = 4096


def _body(s_ref, prog_ref, emb_ref, gain_ref, bias_ref, o_ref, coef_ref):
    @pl.when(pl.program_id(0) == 0)
    def _compute_coefficients():
        w = jax.nn.softmax(prog_ref[...], axis=-1)
        gain = jnp.dot(w, gain_ref[...],
                       preferred_element_type=jnp.float32)
        bias = jnp.dot(w, bias_ref[...],
                       preferred_element_type=jnp.float32)
        c = gain * emb_ref[...] + bias
        P_ = gain.shape[0]
        G = gain[0:1, :]
        C = c[0:1, :]
        for i in range(1, P_):
            g = gain[i:i + 1, :]
            G = g * G
            C = g * C + c[i:i + 1, :]
        coef_ref[0:1, :] = G
        coef_ref[1:2, :] = C

    o_ref[...] = s_ref[...] * coef_ref[0:1, :] + coef_ref[1:2, :]


@jax.jit
def kernel(s, prog, emb, lib_gain, lib_bias):
    B, SD = s.shape
    P_, K = prog.shape
    emb_p = emb[:P_]
    grid = (B // _BLOCK_B,)
    return pl.pallas_call(
        _body,
        grid=grid,
        in_specs=[
            pl.BlockSpec((_BLOCK_B, SD), lambda i: (i, 0)),
            pl.BlockSpec((P_, K), lambda i: (0, 0)),
            pl.BlockSpec((P_, SD), lambda i: (0, 0)),
            pl.BlockSpec((K, SD), lambda i: (0, 0)),
            pl.BlockSpec((K, SD), lambda i: (0, 0)),
        ],
        out_specs=pl.BlockSpec((_BLOCK_B, SD), lambda i: (i, 0)),
        out_shape=jax.ShapeDtypeStruct((B, SD), s.dtype),
        scratch_shapes=[pltpu.VMEM((2, SD), jnp.float32)],
    )(s, prog, emb_p, lib_gain, lib_bias)


# speedup vs baseline: 24.3239x; 1.1279x over previous
import functools

import jax
import jax.numpy as jnp
from jax.experimental import pallas as pl
from jax.experimental.pallas import tpu as pltpu

_BLOCK_B = 8192


def _body(s_ref, prog_ref, emb_ref, gain_ref, bias_ref, o_ref, coef_ref):
    @pl.when(pl.program_id(0) == 0)
    def _compute_coefficients():
        w = jax.nn.softmax(prog_ref[...], axis=-1)
        gain = jnp.dot(w, gain_ref[...],
                       preferred_element_type=jnp.float32)
        bias = jnp.dot(w, bias_ref[...],
                       preferred_element_type=jnp.float32)
        c = gain * emb_ref[...] + bias
        P_ = gain.shape[0]
        G = gain[0:1, :]
        C = c[0:1, :]
        for i in range(1, P_):
            g = gain[i:i + 1, :]
            G = g * G
            C = g * C + c[i:i + 1, :]
        coef_ref[0:1, :] = G
        coef_ref[1:2, :] = C

    o_ref[...] = s_ref[...] * coef_ref[0:1, :] + coef_ref[1:2, :]


@jax.jit
def kernel(s, prog, emb, lib_gain, lib_bias):
    B, SD = s.shape
    P_, K = prog.shape
    emb_p = emb[:P_]
    grid = (B // _BLOCK_B,)
    return pl.pallas_call(
        _body,
        grid=grid,
        in_specs=[
            pl.BlockSpec((_BLOCK_B, SD), lambda i: (i, 0)),
            pl.BlockSpec((P_, K), lambda i: (0, 0)),
            pl.BlockSpec((P_, SD), lambda i: (0, 0)),
            pl.BlockSpec((K, SD), lambda i: (0, 0)),
            pl.BlockSpec((K, SD), lambda i: (0, 0)),
        ],
        out_specs=pl.BlockSpec((_BLOCK_B, SD), lambda i: (i, 0)),
        out_shape=jax.ShapeDtypeStruct((B, SD), s.dtype),
        scratch_shapes=[pltpu.VMEM((2, SD), jnp.float32)],
    )(s, prog, emb_p, lib_gain, lib_bias)
